# Initial kernel scaffold; baseline (speedup 1.0000x reference)
import jax
import jax.numpy as jnp
from jax.experimental import pallas as pl


def kernel(confidence, anchors_all, pr):
    raise NotImplementedError("write your pallas kernel here")


# speedup vs baseline: 2.2243x; 2.2243x over previous
import jax
import jax.numpy as jnp
from jax import lax
from jax.experimental import pallas as pl
from jax.experimental.pallas import tpu as pltpu

_LANES = 128
_SUB = 8


def _nms_kernel(conf_ref, anc_ref, pr_ref,
                sc_out, bx_out, cid_out,
                s_ref, y1_ref, x1_ref, y2_ref, x2_ref, area_ref, idx_ref,
                *, max_boxes, score_thr, iou_thr, rows):
    c = pl.program_id(0)
    neg_inf = jnp.float32(-jnp.inf)

    y1_ref[...] = anc_ref[0] + pr_ref[0]
    x1_ref[...] = anc_ref[1] + pr_ref[1]
    y2_ref[...] = anc_ref[2] + pr_ref[2]
    x2_ref[...] = anc_ref[3] + pr_ref[3]
    area_ref[...] = (y2_ref[...] - y1_ref[...]) * (x2_ref[...] - x1_ref[...])

    conf = conf_ref[0]
    s_ref[...] = jnp.where(conf >= score_thr, conf, neg_inf)
    idx_ref[...] = (lax.broadcasted_iota(jnp.int32, (rows, _LANES), 0) * _LANES
                    + lax.broadcasted_iota(jnp.int32, (rows, _LANES), 1))

    sc_out[...] = jnp.zeros_like(sc_out)
    bx_out[...] = jnp.zeros_like(bx_out)
    cid_out[...] = jnp.zeros_like(cid_out)

    lane1 = lax.broadcasted_iota(jnp.int32, (1, 1, _LANES), 2)
    sub8 = lax.broadcasted_iota(jnp.int32, (_SUB, _LANES), 0)
    lane8 = lax.broadcasted_iota(jnp.int32, (_SUB, _LANES), 1)
    j_iota = lax.broadcasted_iota(jnp.int32, (1, 4, 1), 1)

    def body(t, _):
        s = s_ref[...]
        v = jnp.max(s)
        idx = jnp.min(jnp.where(s == v, idx_ref[...], jnp.int32(rows * _LANES)))
        r = idx // _LANES
        l = idx % _LANES
        rbase = pl.multiple_of((r >> 3) << 3, _SUB)
        pick_m = (sub8 == (r & 7)) & (lane8 == l)

        def pick(ref):
            tile = ref[pl.ds(rbase, _SUB), :]
            return jnp.sum(jnp.where(pick_m, tile, 0.0))

        by1 = pick(y1_ref)
        bx1 = pick(x1_ref)
        by2 = pick(y2_ref)
        bx2 = pick(x2_ref)
        keep = v > neg_inf

        iy1 = jnp.maximum(by1, y1_ref[...])
        ix1 = jnp.maximum(bx1, x1_ref[...])
        iy2 = jnp.minimum(by2, y2_ref[...])
        ix2 = jnp.minimum(bx2, x2_ref[...])
        inter = jnp.maximum(iy2 - iy1, 0.0) * jnp.maximum(ix2 - ix1, 0.0)
        area_a = (by2 - by1) * (bx2 - bx1)
        union = area_a + area_ref[...] - inter
        iou = jnp.where(union > 0.0, inter / union, 0.0)
        news = jnp.where(iou > iou_thr, neg_inf, s)
        news = jnp.where(idx_ref[...] == idx, neg_inf, news)
        s_ref[...] = news

        sel = lane1 == t
        sc_out[...] = jnp.where(sel, jnp.where(keep, v, 0.0), sc_out[...])
        cid_out[...] = jnp.where(sel, jnp.where(keep, c + 1, 0), cid_out[...])
        coords = jnp.where(j_iota == 0, by1,
                           jnp.where(j_iota == 1, bx1,
                                     jnp.where(j_iota == 2, by2, bx2)))
        coords = jnp.where(keep, coords, 0.0)
        bx_out[...] = jnp.where(sel, coords, bx_out[...])
        return 0

    lax.fori_loop(0, max_boxes, body, 0)


def _run_nms(confidence, anchors_all, pr, max_boxes, score_thr, iou_thr):
    n, num_classes = confidence.shape
    rows = ((n + _LANES * _SUB - 1) // (_LANES * _SUB)) * _SUB
    n_pad = rows * _LANES

    conf_t = jnp.pad(confidence.T, ((0, 0), (0, n_pad - n)))
    conf_t = conf_t.reshape(num_classes, rows, _LANES)
    anc_t = jnp.pad(anchors_all.T, ((0, 0), (0, n_pad - n))).reshape(4, rows, _LANES)
    pr_t = jnp.pad(pr.T, ((0, 0), (0, n_pad - n))).reshape(4, rows, _LANES)

    out_lanes = ((max_boxes + _LANES - 1) // _LANES) * _LANES

    def kern(*refs):
        return _nms_kernel(*refs, max_boxes=max_boxes, score_thr=score_thr,
                           iou_thr=iou_thr, rows=rows)

    sc, bx, cid = pl.pallas_call(
        kern,
        grid=(num_classes,),
        in_specs=[
            pl.BlockSpec((1, rows, _LANES), lambda c: (c, 0, 0)),
            pl.BlockSpec((4, rows, _LANES), lambda c: (0, 0, 0)),
            pl.BlockSpec((4, rows, _LANES), lambda c: (0, 0, 0)),
        ],
        out_specs=[
            pl.BlockSpec((1, 1, out_lanes), lambda c: (c, 0, 0)),
            pl.BlockSpec((1, 4, out_lanes), lambda c: (c, 0, 0)),
            pl.BlockSpec((1, 1, out_lanes), lambda c: (c, 0, 0)),
        ],
        out_shape=[
            jax.ShapeDtypeStruct((num_classes, 1, out_lanes), jnp.float32),
            jax.ShapeDtypeStruct((num_classes, 4, out_lanes), jnp.float32),
            jax.ShapeDtypeStruct((num_classes, 1, out_lanes), jnp.int32),
        ],
        scratch_shapes=[
            pltpu.VMEM((rows, _LANES), jnp.float32),
            pltpu.VMEM((rows, _LANES), jnp.float32),
            pltpu.VMEM((rows, _LANES), jnp.float32),
            pltpu.VMEM((rows, _LANES), jnp.float32),
            pltpu.VMEM((rows, _LANES), jnp.float32),
            pltpu.VMEM((rows, _LANES), jnp.float32),
            pltpu.VMEM((rows, _LANES), jnp.int32),
        ],
        compiler_params=pltpu.CompilerParams(
            dimension_semantics=("parallel",),
        ),
    )(conf_t, anc_t, pr_t)

    sel_scores = sc[:, 0, :max_boxes]
    sel_boxes = jnp.transpose(bx[:, :, :max_boxes], (0, 2, 1))
    class_id = cid[:, 0, :max_boxes]
    return sel_scores, sel_boxes, class_id


def kernel(confidence, anchors_all, pr):
    return _run_nms(confidence, anchors_all, pr,
                    max_boxes=100, score_thr=0.5, iou_thr=0.45)


# speedup vs baseline: 2.6262x; 1.1807x over previous
import jax
import jax.numpy as jnp
from jax import lax
from jax.experimental import pallas as pl
from jax.experimental.pallas import tpu as pltpu

_LANES = 128
_SUB = 8


def _nms_kernel(conf_ref, anc_ref, pr_ref,
                sc_out, bx_out, cid_out,
                s_ref, y1_ref, x1_ref, y2_ref, x2_ref, area_ref, idx_ref,
                *, max_boxes, score_thr, iou_thr, rows, pair):
    c = pl.program_id(0)
    neg_inf = jnp.float32(-jnp.inf)
    big = jnp.int32(rows * _LANES)

    y1_ref[...] = anc_ref[0] + pr_ref[0]
    x1_ref[...] = anc_ref[1] + pr_ref[1]
    y2_ref[...] = anc_ref[2] + pr_ref[2]
    x2_ref[...] = anc_ref[3] + pr_ref[3]
    area_ref[...] = (y2_ref[...] - y1_ref[...]) * (x2_ref[...] - x1_ref[...])
    idx_ref[...] = (lax.broadcasted_iota(jnp.int32, (rows, _LANES), 0) * _LANES
                    + lax.broadcasted_iota(jnp.int32, (rows, _LANES), 1))

    sc_out[...] = jnp.zeros_like(sc_out)
    bx_out[...] = jnp.zeros_like(bx_out)
    cid_out[...] = jnp.zeros_like(cid_out)

    lane1 = lax.broadcasted_iota(jnp.int32, (1, 1, _LANES), 2)
    sub8 = lax.broadcasted_iota(jnp.int32, (_SUB, _LANES), 0)
    lane8 = lax.broadcasted_iota(jnp.int32, (_SUB, _LANES), 1)
    j_iota = lax.broadcasted_iota(jnp.int32, (1, 4, 1), 1)

    def first_argmax(p):
        s0 = jnp.where(conf_ref[p] >= score_thr, conf_ref[p], neg_inf)
        s_ref[p] = s0
        colmax = jnp.max(s0, axis=0, keepdims=True)
        rowhit = jnp.min(jnp.where(s0 == colmax, idx_ref[...], big),
                         axis=0, keepdims=True)
        v = jnp.max(colmax)
        idx = jnp.min(jnp.where(colmax == v, rowhit, big))
        return v, idx

    init = tuple(first_argmax(p) for p in range(pair))
    init = tuple(x for vi in init for x in vi)

    def one_class(p, t, v, idx):
        r = idx // _LANES
        l = idx % _LANES
        rbase = pl.multiple_of((r >> 3) << 3, _SUB)
        pick_m = (sub8 == (r & 7)) & (lane8 == l)

        def pick(ref):
            tile = ref[pl.ds(rbase, _SUB), :]
            return jnp.sum(jnp.where(pick_m, tile, 0.0))

        by1 = pick(y1_ref)
        bx1 = pick(x1_ref)
        by2 = pick(y2_ref)
        bx2 = pick(x2_ref)
        keep = v > neg_inf

        s = s_ref[p]
        iy1 = jnp.maximum(by1, y1_ref[...])
        ix1 = jnp.maximum(bx1, x1_ref[...])
        iy2 = jnp.minimum(by2, y2_ref[...])
        ix2 = jnp.minimum(bx2, x2_ref[...])
        inter = jnp.maximum(iy2 - iy1, 0.0) * jnp.maximum(ix2 - ix1, 0.0)
        area_a = (by2 - by1) * (bx2 - bx1)
        union = area_a + area_ref[...] - inter
        iou = jnp.where(union > 0.0, inter / union, 0.0)
        news = jnp.where((iou > iou_thr) | (idx_ref[...] == idx), neg_inf, s)
        s_ref[p] = news

        colmax = jnp.max(news, axis=0, keepdims=True)
        rowhit = jnp.min(jnp.where(news == colmax, idx_ref[...], big),
                         axis=0, keepdims=True)
        v2 = jnp.max(colmax)
        idx2 = jnp.min(jnp.where(colmax == v2, rowhit, big))

        sel = lane1 == t
        sc_out[p] = jnp.where(sel[0], jnp.where(keep, v, 0.0), sc_out[p])
        cid_out[p] = jnp.where(sel[0], jnp.where(keep, c * pair + p + 1, 0),
                               cid_out[p])
        coords = jnp.where(j_iota == 0, by1,
                           jnp.where(j_iota == 1, bx1,
                                     jnp.where(j_iota == 2, by2, bx2)))
        coords = jnp.where(keep, coords, 0.0)
        bx_out[pl.ds(p, 1)] = jnp.where(sel, coords, bx_out[pl.ds(p, 1)])
        return v2, idx2

    def body(t, carry):
        out = []
        for p in range(pair):
            v, idx = carry[2 * p], carry[2 * p + 1]
            out.extend(one_class(p, t, v, idx))
        return tuple(out)

    lax.fori_loop(0, max_boxes, body, init)


def _run_nms(confidence, anchors_all, pr, max_boxes, score_thr, iou_thr):
    n, num_classes = confidence.shape
    rows = ((n + _LANES * _SUB - 1) // (_LANES * _SUB)) * _SUB
    n_pad = rows * _LANES
    pair = 2 if num_classes % 2 == 0 else 1

    conf_t = jnp.pad(confidence.T, ((0, 0), (0, n_pad - n)))
    conf_t = conf_t.reshape(num_classes, rows, _LANES)
    anc_t = jnp.pad(anchors_all.T, ((0, 0), (0, n_pad - n))).reshape(4, rows, _LANES)
    pr_t = jnp.pad(pr.T, ((0, 0), (0, n_pad - n))).reshape(4, rows, _LANES)

    out_lanes = ((max_boxes + _LANES - 1) // _LANES) * _LANES

    def kern(*refs):
        return _nms_kernel(*refs, max_boxes=max_boxes, score_thr=score_thr,
                           iou_thr=iou_thr, rows=rows, pair=pair)

    sc, bx, cid = pl.pallas_call(
        kern,
        grid=(num_classes // pair,),
        in_specs=[
            pl.BlockSpec((pair, rows, _LANES), lambda c: (c, 0, 0)),
            pl.BlockSpec((4, rows, _LANES), lambda c: (0, 0, 0)),
            pl.BlockSpec((4, rows, _LANES), lambda c: (0, 0, 0)),
        ],
        out_specs=[
            pl.BlockSpec((pair, 1, out_lanes), lambda c: (c, 0, 0)),
            pl.BlockSpec((pair, 4, out_lanes), lambda c: (c, 0, 0)),
            pl.BlockSpec((pair, 1, out_lanes), lambda c: (c, 0, 0)),
        ],
        out_shape=[
            jax.ShapeDtypeStruct((num_classes, 1, out_lanes), jnp.float32),
            jax.ShapeDtypeStruct((num_classes, 4, out_lanes), jnp.float32),
            jax.ShapeDtypeStruct((num_classes, 1, out_lanes), jnp.int32),
        ],
        scratch_shapes=[
            pltpu.VMEM((pair, rows, _LANES), jnp.float32),
            pltpu.VMEM((rows, _LANES), jnp.float32),
            pltpu.VMEM((rows, _LANES), jnp.float32),
            pltpu.VMEM((rows, _LANES), jnp.float32),
            pltpu.VMEM((rows, _LANES), jnp.float32),
            pltpu.VMEM((rows, _LANES), jnp.float32),
            pltpu.VMEM((rows, _LANES), jnp.int32),
        ],
        compiler_params=pltpu.CompilerParams(
            dimension_semantics=("parallel",),
        ),
    )(conf_t, anc_t, pr_t)

    sel_scores = sc[:, 0, :max_boxes]
    sel_boxes = jnp.transpose(bx[:, :, :max_boxes], (0, 2, 1))
    class_id = cid[:, 0, :max_boxes]
    return sel_scores, sel_boxes, class_id


def kernel(confidence, anchors_all, pr):
    return _run_nms(confidence, anchors_all, pr,
                    max_boxes=100, score_thr=0.5, iou_thr=0.45)
